# baseline (speedup 1.0000x reference)
import jax
import jax.numpy as jnp
from jax.experimental import pallas as pl
from jax.experimental.pallas import tpu as pltpu

N_POINTS = 500000
VOXEL_SIZE = 0.1
DT_PREDICTION = 0.1
HIDDEN = 64

_BR = 2500
_GRID = N_POINTS // _BR
_NBUF = 8


def _body(q_ref, w1_ref, b1_ref, w2_ref, b2_ref, pc_hbm, oc_hbm, of_hbm,
          xv, ocv, ofv, insem, ocsem, ofsem):
    h = jnp.maximum(w1_ref[...] * 0.5 + b1_ref[...], 0.0)
    s = jnp.sum(h * w2_ref[...]) + b2_ref[0, 0]
    ofv[...] = jnp.full(ofv.shape, s, dtype=ofv.dtype)
    q = q_ref[...]

    def in_copy(j, slot):
        return pltpu.make_async_copy(
            pc_hbm.at[pl.ds(j * _BR, _BR), :], xv.at[slot], insem.at[slot])

    def oc_copy(j, slot):
        return pltpu.make_async_copy(
            ocv.at[slot], oc_hbm.at[pl.ds(j * _BR, _BR), :], ocsem.at[slot])

    def of_copy(j, slot):
        return pltpu.make_async_copy(
            ofv, of_hbm.at[pl.ds(j * _BR, _BR), :], ofsem.at[slot])

    def compute(j, slot):
        in_copy(j, slot).wait()
        ocv[slot] = (xv[slot] / q) * q
        oc_copy(j, slot).start()
        of_copy(j, slot).start()

    for k in range(_NBUF):
        in_copy(k, k).start()

    for j in range(_NBUF):
        compute(j, j)
        if j + _NBUF < _GRID:
            in_copy(j + _NBUF, j).start()

    def step(j, _):
        slot = jax.lax.rem(j, _NBUF)
        oc_copy(j - _NBUF, slot).wait()
        of_copy(j - _NBUF, slot).wait()
        compute(j, slot)
        in_copy(j + _NBUF, slot).start()
        return 0

    jax.lax.fori_loop(_NBUF, _GRID - _NBUF, step, 0)

    for j in range(_GRID - _NBUF, _GRID):
        slot = j % _NBUF
        oc_copy(j - _NBUF, slot).wait()
        of_copy(j - _NBUF, slot).wait()
        compute(j, slot)

    for j in range(_GRID - _NBUF, _GRID):
        slot = j % _NBUF
        oc_copy(j, slot).wait()
        of_copy(j, slot).wait()


def kernel(point_cloud, W1, b1, W2, b2):
    qrow = jnp.array([[VOXEL_SIZE, VOXEL_SIZE, VOXEL_SIZE, DT_PREDICTION]],
                     dtype=point_cloud.dtype)
    w1 = W1.reshape(1, HIDDEN)
    b1r = b1.reshape(1, HIDDEN)
    w2 = W2.reshape(1, HIDDEN)
    b2r = b2.reshape(1, 1)

    vmem = lambda shape: pl.BlockSpec(shape, lambda: (0, 0))
    out_coords, out_feats = pl.pallas_call(
        _body,
        in_specs=[
            vmem((1, 4)),
            vmem((1, HIDDEN)),
            vmem((1, HIDDEN)),
            vmem((1, HIDDEN)),
            vmem((1, 1)),
            pl.BlockSpec(memory_space=pltpu.HBM),
        ],
        out_specs=[
            pl.BlockSpec(memory_space=pltpu.HBM),
            pl.BlockSpec(memory_space=pltpu.HBM),
        ],
        out_shape=[
            jax.ShapeDtypeStruct((N_POINTS, 4), point_cloud.dtype),
            jax.ShapeDtypeStruct((N_POINTS, 1), point_cloud.dtype),
        ],
        scratch_shapes=[
            pltpu.VMEM((_NBUF, _BR, 4), jnp.float32),
            pltpu.VMEM((_NBUF, _BR, 4), jnp.float32),
            pltpu.VMEM((_BR, 1), jnp.float32),
            pltpu.SemaphoreType.DMA((_NBUF,)),
            pltpu.SemaphoreType.DMA((_NBUF,)),
            pltpu.SemaphoreType.DMA((_NBUF,)),
        ],
    )(qrow, w1, b1r, w2, b2r, point_cloud)
    return out_feats, out_coords


# speedup vs baseline: 1.0065x; 1.0065x over previous
import jax
import jax.numpy as jnp
from jax.experimental import pallas as pl

N_POINTS = 500000
VOXEL_SIZE = 0.1
DT_PREDICTION = 0.1
HIDDEN = 64

_BR = 5000
_GRID = N_POINTS // _BR


def _body(x_ref, q_ref, w1_ref, b1_ref, w2_ref, b2_ref, oc_ref, of_ref):
    q = q_ref[...]
    oc_ref[...] = (x_ref[...] / q) * q
    h = jnp.maximum(w1_ref[...] * 0.5 + b1_ref[...], 0.0)
    s = jnp.sum(h * w2_ref[...]) + b2_ref[0, 0]
    of_ref[...] = jnp.full(of_ref.shape, s, dtype=of_ref.dtype)


def kernel(point_cloud, W1, b1, W2, b2):
    qrow = jnp.array([[VOXEL_SIZE, VOXEL_SIZE, VOXEL_SIZE, DT_PREDICTION]],
                     dtype=point_cloud.dtype)
    w1 = W1.reshape(1, HIDDEN)
    b1r = b1.reshape(1, HIDDEN)
    w2 = W2.reshape(1, HIDDEN)
    b2r = b2.reshape(1, 1)

    full = lambda shape: pl.BlockSpec(shape, lambda i: (0, 0))
    out_coords, out_feats = pl.pallas_call(
        _body,
        grid=(_GRID,),
        in_specs=[
            pl.BlockSpec((_BR, 4), lambda i: (i, 0)),
            full((1, 4)),
            full((1, HIDDEN)),
            full((1, HIDDEN)),
            full((1, HIDDEN)),
            full((1, 1)),
        ],
        out_specs=[
            pl.BlockSpec((_BR, 4), lambda i: (i, 0)),
            pl.BlockSpec((_BR, 1), lambda i: (i, 0)),
        ],
        out_shape=[
            jax.ShapeDtypeStruct((N_POINTS, 4), point_cloud.dtype),
            jax.ShapeDtypeStruct((N_POINTS, 1), point_cloud.dtype),
        ],
    )(point_cloud, qrow, w1, b1r, w2, b2r)
    return out_feats, out_coords


# speedup vs baseline: 1.0181x; 1.0115x over previous
import jax
import jax.numpy as jnp
from jax.experimental import pallas as pl

N_POINTS = 500000
VOXEL_SIZE = 0.1
DT_PREDICTION = 0.1
HIDDEN = 64

_BR = 10000
_GRID = N_POINTS // _BR


def _body(x_ref, q_ref, w1_ref, b1_ref, w2_ref, b2_ref, oc_ref, of_ref):
    q = q_ref[...]
    oc_ref[...] = (x_ref[...] / q) * q
    h = jnp.maximum(w1_ref[...] * 0.5 + b1_ref[...], 0.0)
    s = jnp.sum(h * w2_ref[...]) + b2_ref[0, 0]
    of_ref[...] = jnp.full(of_ref.shape, s, dtype=of_ref.dtype)


def kernel(point_cloud, W1, b1, W2, b2):
    qrow = jnp.array([[VOXEL_SIZE, VOXEL_SIZE, VOXEL_SIZE, DT_PREDICTION]],
                     dtype=point_cloud.dtype)
    w1 = W1.reshape(1, HIDDEN)
    b1r = b1.reshape(1, HIDDEN)
    w2 = W2.reshape(1, HIDDEN)
    b2r = b2.reshape(1, 1)

    full = lambda shape: pl.BlockSpec(shape, lambda i: (0, 0))
    out_coords, out_feats = pl.pallas_call(
        _body,
        grid=(_GRID,),
        in_specs=[
            pl.BlockSpec((_BR, 4), lambda i: (i, 0)),
            full((1, 4)),
            full((1, HIDDEN)),
            full((1, HIDDEN)),
            full((1, HIDDEN)),
            full((1, 1)),
        ],
        out_specs=[
            pl.BlockSpec((_BR, 4), lambda i: (i, 0)),
            pl.BlockSpec((_BR, 1), lambda i: (i, 0)),
        ],
        out_shape=[
            jax.ShapeDtypeStruct((N_POINTS, 4), point_cloud.dtype),
            jax.ShapeDtypeStruct((N_POINTS, 1), point_cloud.dtype),
        ],
    )(point_cloud, qrow, w1, b1r, w2, b2r)
    return out_feats, out_coords


# speedup vs baseline: 1.0182x; 1.0002x over previous
import jax
import jax.numpy as jnp
import jax
from jax.experimental import pallas as pl

N_POINTS = 500000
VOXEL_SIZE = 0.1
DT_PREDICTION = 0.1
HIDDEN = 64

_BR = 10000
_GRID = N_POINTS // _BR


def _body(x_ref, q_ref, w1_ref, b1_ref, w2_ref, b2_ref, oc_ref, of_ref):
    q = q_ref[...]
    oc_ref[...] = (x_ref[...] / q) * q
    bf = lambda v: v.astype(jnp.bfloat16).astype(jnp.float32)
    h = jnp.maximum(0.5 * bf(w1_ref[...]) + b1_ref[...], 0.0)
    s = jnp.sum(bf(h) * bf(w2_ref[...])) + b2_ref[0, 0]
    of_ref[...] = jnp.full(of_ref.shape, s, dtype=of_ref.dtype)


def kernel(point_cloud, W1, b1, W2, b2):
    qrow = jnp.array([[VOXEL_SIZE, VOXEL_SIZE, VOXEL_SIZE, DT_PREDICTION]],
                     dtype=point_cloud.dtype)
    w1 = W1.reshape(1, HIDDEN)
    b1r = b1.reshape(1, HIDDEN)
    w2 = W2.reshape(1, HIDDEN)
    b2r = b2.reshape(1, 1)

    full = lambda shape: pl.BlockSpec(shape, lambda i: (0, 0))
    out_coords, out_feats = pl.pallas_call(
        _body,
        grid=(_GRID,),
        in_specs=[
            pl.BlockSpec((_BR, 4), lambda i: (i, 0)),
            full((1, 4)),
            full((1, HIDDEN)),
            full((1, HIDDEN)),
            full((1, HIDDEN)),
            full((1, 1)),
        ],
        out_specs=[
            pl.BlockSpec((_BR, 4), lambda i: (i, 0)),
            pl.BlockSpec((_BR, 1), lambda i: (i, 0)),
        ],
        out_shape=[
            jax.ShapeDtypeStruct((N_POINTS, 4), point_cloud.dtype),
            jax.ShapeDtypeStruct((N_POINTS, 1), point_cloud.dtype),
        ],
    )(point_cloud, qrow, w1, b1r, w2, b2r)
    return out_feats, out_coords
